# baseline (speedup 1.0000x reference)
import functools

import jax
import jax.numpy as jnp
from jax import lax
from jax.experimental import pallas as pl
from jax.experimental.pallas import tpu as pltpu
from jax.experimental.pallas import tpu_sc as plsc

_BATCH = 16384
_EMBED = 64
_PACK = 128
_ROWS = 1000000
_PROWS = _ROWS // 2
_NFULL = _ROWS // _PACK
_TAIL = _ROWS - _NFULL * _PACK
_NUM_CORES = 2
_NUM_SUBCORES = 16
_NW = _NUM_CORES * _NUM_SUBCORES
_BPW = _BATCH // _NW
_CHUNK = 128
_NCHUNK = _BPW // _CHUNK
_TRIPS = (_NFULL + _NW - 1) // _NW
_NPAIR = (_TRIPS + 1) // 2

_mesh = plsc.VectorSubcoreMesh(core_axis_name="c", subcore_axis_name="s")


@functools.partial(
    pl.kernel,
    mesh=_mesh,
    compiler_params=pltpu.CompilerParams(needs_layout_passes=False),
    out_type=(jax.ShapeDtypeStruct((_PROWS, _PACK), jnp.float32),
              jax.ShapeDtypeStruct((_PROWS, _PACK), jnp.float32)),
    scratch_types=[
        pltpu.VMEM((2, 2, _EMBED, _PACK), jnp.float32),
        pltpu.VMEM((2, 2, _EMBED, _PACK), jnp.float32),
        pltpu.SemaphoreType.DMA,
        pltpu.SemaphoreType.DMA,
        pltpu.SemaphoreType.DMA,
        pltpu.SemaphoreType.DMA,
    ],
)
def _repack(ut_hbm, it_hbm, utail_hbm, itail_hbm, up_hbm, ip_hbm, src, dst,
            sem_in0, sem_in1, sem_out0, sem_out1):
    wid = lax.axis_index("s") * _NUM_CORES + lax.axis_index("c")
    sems_in = (sem_in0, sem_in1)
    sems_out = (sem_out0, sem_out1)
    tbls_in = (ut_hbm, it_hbm)
    tbls_out = (up_hbm, ip_hbm)

    lanes = lax.iota(jnp.int32, 16)
    k16 = [lanes + 16 * k for k in range(4)]

    def blk_of(t):
        return jnp.clip(t * _NW + wid, 0, _NFULL - 1)

    def guard(t):
        return (t * _NW + wid) < _NFULL

    def in_copies(t, phase):
        blk = blk_of(t)
        cps = []
        for tb in range(2):
            for ch in range(8):
                cps.append(pltpu.make_async_copy(
                    tbls_in[tb].at[pl.ds(ch * 8, 8),
                                   pl.ds(blk * _PACK, _PACK)],
                    src.at[phase, tb, pl.ds(ch * 8, 8)],
                    sems_in[phase]))
        return cps

    def out_copies(t, phase):
        blk = blk_of(t)
        cps = []
        for tb in range(2):
            cps.append(pltpu.make_async_copy(
                dst.at[phase, tb],
                tbls_out[tb].at[pl.ds(blk * (_PACK // 2), _PACK // 2)],
                sems_out[phase]))
        return cps

    def transpose(sref, dref, npack):
        for pm in range(npack):
            for h in range(2):
                col = jnp.full((16,), 2 * pm + h, jnp.int32)
                for k4 in range(4):
                    v = plsc.load_gather(sref, [k16[k4], col])
                    dref[pm, pl.ds(h * 64 + k4 * 16, 16)] = v

    @pl.when(guard(0))
    def _():
        for c in in_copies(0, 0):
            c.start()

    def trip_pair(k, carry):
        for half in range(2):
            t = k * 2 + half
            phase = half

            @pl.when(guard(t))
            def _(t=t, phase=phase):
                for c in in_copies(t, phase):
                    c.wait()

            @pl.when(guard(t + 1))
            def _(t=t, phase=phase):
                for c in in_copies(t + 1, 1 - phase):
                    c.start()

            @pl.when(jnp.logical_and(t >= 2, guard(t - 2)))
            def _(t=t, phase=phase):
                for c in out_copies(t - 2, phase):
                    c.wait()

            @pl.when(guard(t))
            def _(t=t, phase=phase):
                for tb in range(2):
                    transpose(src.at[phase, tb], dst.at[phase, tb],
                              _PACK // 2)
                for c in out_copies(t, phase):
                    c.start()
        return carry

    lax.fori_loop(0, _NPAIR, trip_pair, 0)

    for t_last in (2 * _NPAIR - 2, 2 * _NPAIR - 1):
        @pl.when(guard(t_last))
        def _(t_last=t_last):
            for c in out_copies(t_last, t_last % 2):
                c.wait()

    tail_wid = _NFULL % _NW
    tails_in = (utail_hbm, itail_hbm)

    @pl.when(wid == tail_wid)
    def _():
        for tb in range(2):
            buf = src.at[0, tb, pl.ds(0, _TAIL // 2)]
            pltpu.sync_copy(tails_in[tb], buf)
            pltpu.sync_copy(
                buf, tbls_out[tb].at[pl.ds(_NFULL * (_PACK // 2),
                                           _TAIL // 2)])


@functools.partial(
    pl.kernel,
    mesh=_mesh,
    compiler_params=pltpu.CompilerParams(needs_layout_passes=False),
    out_type=jax.ShapeDtypeStruct((_BATCH,), jnp.float32),
    scratch_types=[
        pltpu.VMEM((_NCHUNK, _CHUNK), jnp.int32),
        pltpu.VMEM((_NCHUNK, _CHUNK), jnp.int32),
        pltpu.VMEM((_NCHUNK, _CHUNK), jnp.int32),
        pltpu.VMEM((_NCHUNK, _CHUNK), jnp.int32),
        pltpu.VMEM((2, _CHUNK, _PACK), jnp.float32),
        pltpu.VMEM((2, _CHUNK, _PACK), jnp.float32),
        pltpu.VMEM((_BPW,), jnp.float32),
        pltpu.SemaphoreType.DMA,
        pltpu.SemaphoreType.DMA,
    ],
)
def _gather_dot(uid_hbm, iid_hbm, ut_hbm, it_hbm, out_hbm,
                uid_v, iid_v, urow_v, irow_v, ubuf, ibuf, out_v,
                sem0, sem1):
    wid = lax.axis_index("s") * _NUM_CORES + lax.axis_index("c")
    base = wid * _BPW

    for j in range(_NCHUNK):
        pltpu.sync_copy(uid_hbm.at[pl.ds(base + j * _CHUNK, _CHUNK)],
                        uid_v.at[j])
        pltpu.sync_copy(iid_hbm.at[pl.ds(base + j * _CHUNK, _CHUNK)],
                        iid_v.at[j])

    for j in range(_NCHUNK):
        for s in range(_CHUNK // 16):
            sl = pl.ds(s * 16, 16)
            urow_v[j, sl] = jax.lax.shift_right_logical(uid_v[j, sl], 1)
            irow_v[j, sl] = jax.lax.shift_right_logical(iid_v[j, sl], 1)

    sems = (sem0, sem1)

    def fire(j):
        cu = pltpu.async_copy(ut_hbm.at[urow_v.at[j]], ubuf.at[j % 2],
                              sems[j % 2])
        ci = pltpu.async_copy(it_hbm.at[irow_v.at[j]], ibuf.at[j % 2],
                              sems[j % 2])
        return (cu, ci)

    lanes = lax.iota(jnp.int32, 16)
    inflight = [fire(0), fire(1)]

    for j in range(_NCHUNK):
        cu, ci = inflight[j]
        cu.wait()
        ci.wait()

        ub = ubuf.at[j % 2]
        ib = ibuf.at[j % 2]

        def group_body(g, carry, j=j, ub=ub, ib=ib):
            sl = pl.ds(g * 16, 16)
            row_idx = g * 16 + lanes
            ucol = jax.lax.bitwise_and(uid_v[j, sl], 1) * _EMBED
            icol = jax.lax.bitwise_and(iid_v[j, sl], 1) * _EMBED
            acc = jnp.zeros((16,), jnp.float32)
            for c in range(_EMBED):
                u = plsc.load_gather(ub, [row_idx, ucol + c])
                v = plsc.load_gather(ib, [row_idx, icol + c])
                acc = acc + u * v
            out_v[pl.ds(j * _CHUNK + g * 16, 16)] = acc
            return carry

        lax.fori_loop(0, _CHUNK // 16, group_body, 0)

        if j + 2 < _NCHUNK:
            inflight.append(fire(j + 2))

    pltpu.sync_copy(out_v, out_hbm.at[pl.ds(base, _BPW)])


def kernel(user_ids, item_ids, user_table, item_table):
    ut_tail = user_table[_ROWS - _TAIL:].reshape(_TAIL // 2, _PACK)
    it_tail = item_table[_ROWS - _TAIL:].reshape(_TAIL // 2, _PACK)
    up, ip = _repack(user_table.T, item_table.T, ut_tail, it_tail)
    return _gather_dot(user_ids, item_ids, up, ip)


# speedup vs baseline: 1.1630x; 1.1630x over previous
import functools

import jax
import jax.numpy as jnp
from jax import lax
from jax.experimental import pallas as pl
from jax.experimental.pallas import tpu as pltpu
from jax.experimental.pallas import tpu_sc as plsc

_BATCH = 16384
_EMBED = 64
_PACK = 128
_ROWS = 1000000
_PROWS = _ROWS // 2
_NFULL = _ROWS // _PACK
_TAIL = _ROWS - _NFULL * _PACK
_NUM_CORES = 2
_NUM_SUBCORES = 16
_NW = _NUM_CORES * _NUM_SUBCORES
_BPW = _BATCH // _NW
_CHUNK = 128
_NCHUNK = _BPW // _CHUNK
_TRIPS = (_NFULL + _NW - 1) // _NW
_NPAIR = (_TRIPS + 1) // 2

_mesh = plsc.VectorSubcoreMesh(core_axis_name="c", subcore_axis_name="s")


@functools.partial(
    pl.kernel,
    mesh=_mesh,
    compiler_params=pltpu.CompilerParams(needs_layout_passes=False),
    out_type=(jax.ShapeDtypeStruct((_PROWS, _PACK), jnp.float32),
              jax.ShapeDtypeStruct((_PROWS, _PACK), jnp.float32)),
    scratch_types=[
        pltpu.VMEM((2, 2, _EMBED, _PACK), jnp.float32),
        pltpu.VMEM((2, 2, _EMBED, _PACK), jnp.float32),
        pltpu.SemaphoreType.DMA,
        pltpu.SemaphoreType.DMA,
        pltpu.SemaphoreType.DMA,
        pltpu.SemaphoreType.DMA,
    ],
)
def _repack(ut_hbm, it_hbm, utail_hbm, itail_hbm, up_hbm, ip_hbm, src, dst,
            sem_in0, sem_in1, sem_out0, sem_out1):
    wid = lax.axis_index("s") * _NUM_CORES + lax.axis_index("c")
    sems_in = (sem_in0, sem_in1)
    sems_out = (sem_out0, sem_out1)
    tbls_in = (ut_hbm, it_hbm)
    tbls_out = (up_hbm, ip_hbm)

    lanes = lax.iota(jnp.int32, 16)
    k16 = [lanes + 16 * k for k in range(4)]

    def blk_of(t):
        return jnp.clip(t * _NW + wid, 0, _NFULL - 1)

    def guard(t):
        return (t * _NW + wid) < _NFULL

    def in_copies(t, phase):
        blk = blk_of(t)
        cps = []
        for tb in range(2):
            for ch in range(8):
                cps.append(pltpu.make_async_copy(
                    tbls_in[tb].at[pl.ds(ch * 8, 8),
                                   pl.ds(blk * _PACK, _PACK)],
                    src.at[phase, tb, pl.ds(ch * 8, 8)],
                    sems_in[phase]))
        return cps

    def out_copies(t, phase):
        blk = blk_of(t)
        cps = []
        for tb in range(2):
            cps.append(pltpu.make_async_copy(
                dst.at[phase, tb],
                tbls_out[tb].at[pl.ds(blk * (_PACK // 2), _PACK // 2)],
                sems_out[phase]))
        return cps

    def transpose(sref, dref, npack):
        def pm_body(pm, carry):
            for h in range(2):
                col = jnp.broadcast_to(2 * pm + h, (16,)).astype(jnp.int32)
                for k4 in range(4):
                    v = plsc.load_gather(sref, [k16[k4], col])
                    dref[pm, pl.ds(h * 64 + k4 * 16, 16)] = v
            return carry

        lax.fori_loop(0, npack, pm_body, 0)

    @pl.when(guard(0))
    def _():
        for c in in_copies(0, 0):
            c.start()

    def trip_pair(k, carry):
        for half in range(2):
            t = k * 2 + half
            phase = half

            @pl.when(guard(t))
            def _(t=t, phase=phase):
                for c in in_copies(t, phase):
                    c.wait()

            @pl.when(guard(t + 1))
            def _(t=t, phase=phase):
                for c in in_copies(t + 1, 1 - phase):
                    c.start()

            @pl.when(jnp.logical_and(t >= 2, guard(t - 2)))
            def _(t=t, phase=phase):
                for c in out_copies(t - 2, phase):
                    c.wait()

            @pl.when(guard(t))
            def _(t=t, phase=phase):
                for tb in range(2):
                    transpose(src.at[phase, tb], dst.at[phase, tb],
                              _PACK // 2)
                for c in out_copies(t, phase):
                    c.start()
        return carry

    lax.fori_loop(0, _NPAIR, trip_pair, 0)

    for t_last in (2 * _NPAIR - 2, 2 * _NPAIR - 1):
        @pl.when(guard(t_last))
        def _(t_last=t_last):
            for c in out_copies(t_last, t_last % 2):
                c.wait()

    tail_wid = _NFULL % _NW
    tails_in = (utail_hbm, itail_hbm)

    @pl.when(wid == tail_wid)
    def _():
        for tb in range(2):
            buf = src.at[0, tb, pl.ds(0, _TAIL // 2)]
            pltpu.sync_copy(tails_in[tb], buf)
            pltpu.sync_copy(
                buf, tbls_out[tb].at[pl.ds(_NFULL * (_PACK // 2),
                                           _TAIL // 2)])


@functools.partial(
    pl.kernel,
    mesh=_mesh,
    compiler_params=pltpu.CompilerParams(needs_layout_passes=False),
    out_type=jax.ShapeDtypeStruct((_BATCH,), jnp.float32),
    scratch_types=[
        pltpu.VMEM((_NCHUNK, _CHUNK), jnp.int32),
        pltpu.VMEM((_NCHUNK, _CHUNK), jnp.int32),
        pltpu.VMEM((_NCHUNK, _CHUNK), jnp.int32),
        pltpu.VMEM((_NCHUNK, _CHUNK), jnp.int32),
        pltpu.VMEM((2, _CHUNK, _PACK), jnp.float32),
        pltpu.VMEM((2, _CHUNK, _PACK), jnp.float32),
        pltpu.VMEM((_BPW,), jnp.float32),
        pltpu.SemaphoreType.DMA,
        pltpu.SemaphoreType.DMA,
    ],
)
def _gather_dot(uid_hbm, iid_hbm, ut_hbm, it_hbm, out_hbm,
                uid_v, iid_v, urow_v, irow_v, ubuf, ibuf, out_v,
                sem0, sem1):
    wid = lax.axis_index("s") * _NUM_CORES + lax.axis_index("c")
    base = wid * _BPW

    for j in range(_NCHUNK):
        pltpu.sync_copy(uid_hbm.at[pl.ds(base + j * _CHUNK, _CHUNK)],
                        uid_v.at[j])
        pltpu.sync_copy(iid_hbm.at[pl.ds(base + j * _CHUNK, _CHUNK)],
                        iid_v.at[j])

    for j in range(_NCHUNK):
        for s in range(_CHUNK // 16):
            sl = pl.ds(s * 16, 16)
            urow_v[j, sl] = jax.lax.shift_right_logical(uid_v[j, sl], 1)
            irow_v[j, sl] = jax.lax.shift_right_logical(iid_v[j, sl], 1)

    sems = (sem0, sem1)

    def fire(j):
        cu = pltpu.async_copy(ut_hbm.at[urow_v.at[j]], ubuf.at[j % 2],
                              sems[j % 2])
        ci = pltpu.async_copy(it_hbm.at[irow_v.at[j]], ibuf.at[j % 2],
                              sems[j % 2])
        return (cu, ci)

    lanes = lax.iota(jnp.int32, 16)
    inflight = [fire(0), fire(1)]

    for j in range(_NCHUNK):
        cu, ci = inflight[j]
        cu.wait()
        ci.wait()

        ub = ubuf.at[j % 2]
        ib = ibuf.at[j % 2]

        def group_body(g, carry, j=j, ub=ub, ib=ib):
            sl = pl.ds(g * 16, 16)
            row_idx = g * 16 + lanes
            ucol = jax.lax.bitwise_and(uid_v[j, sl], 1) * _EMBED
            icol = jax.lax.bitwise_and(iid_v[j, sl], 1) * _EMBED
            acc = jnp.zeros((16,), jnp.float32)
            for c in range(_EMBED):
                u = plsc.load_gather(ub, [row_idx, ucol + c])
                v = plsc.load_gather(ib, [row_idx, icol + c])
                acc = acc + u * v
            out_v[pl.ds(j * _CHUNK + g * 16, 16)] = acc
            return carry

        lax.fori_loop(0, _CHUNK // 16, group_body, 0)

        if j + 2 < _NCHUNK:
            inflight.append(fire(j + 2))

    pltpu.sync_copy(out_v, out_hbm.at[pl.ds(base, _BPW)])


def kernel(user_ids, item_ids, user_table, item_table):
    ut_tail = user_table[_ROWS - _TAIL:].reshape(_TAIL // 2, _PACK)
    it_tail = item_table[_ROWS - _TAIL:].reshape(_TAIL // 2, _PACK)
    up, ip = _repack(user_table.T, item_table.T, ut_tail, it_tail)
    return _gather_dot(user_ids, item_ids, up, ip)


# speedup vs baseline: 3.2217x; 2.7701x over previous
import functools

import jax
import jax.numpy as jnp
from jax import lax
from jax.experimental import pallas as pl
from jax.experimental.pallas import tpu as pltpu
from jax.experimental.pallas import tpu_sc as plsc

_BATCH = 16384
_EMBED = 64
_NUM_CORES = 2
_NUM_SUBCORES = 16
_NW = _NUM_CORES * _NUM_SUBCORES
_BPW = _BATCH // _NW
_CHUNK = 128
_NCHUNK = _BPW // _CHUNK

_mesh = plsc.VectorSubcoreMesh(core_axis_name="c", subcore_axis_name="s")


@functools.partial(
    pl.kernel,
    mesh=_mesh,
    compiler_params=pltpu.CompilerParams(
        needs_layout_passes=False, use_tc_tiling_on_sc=False),
    out_type=jax.ShapeDtypeStruct((_BATCH,), jnp.float32),
    scratch_types=[
        pltpu.VMEM((_NCHUNK, _CHUNK), jnp.int32),
        pltpu.VMEM((_NCHUNK, _CHUNK), jnp.int32),
        pltpu.VMEM((_BPW, _EMBED), jnp.float32),
        pltpu.VMEM((_BPW, _EMBED), jnp.float32),
        pltpu.VMEM((_BPW,), jnp.float32),
        pltpu.SemaphoreType.DMA,
    ],
)
def _sc_kernel(uid_hbm, iid_hbm, ut_hbm, it_hbm, out_hbm,
               uid_v, iid_v, urows, irows, out_v, sem):
    wid = lax.axis_index("s") * _NUM_CORES + lax.axis_index("c")
    base = wid * _BPW

    for j in range(_NCHUNK):
        pltpu.sync_copy(uid_hbm.at[pl.ds(base + j * _CHUNK, _CHUNK)],
                        uid_v.at[j])
        pltpu.sync_copy(iid_hbm.at[pl.ds(base + j * _CHUNK, _CHUNK)],
                        iid_v.at[j])

    copies = []
    for j in range(_NCHUNK):
        copies.append(pltpu.async_copy(
            ut_hbm.at[uid_v.at[j]], urows.at[pl.ds(j * _CHUNK, _CHUNK)], sem))
        copies.append(pltpu.async_copy(
            it_hbm.at[iid_v.at[j]], irows.at[pl.ds(j * _CHUNK, _CHUNK)], sem))
    for c in copies:
        c.wait()

    lane = lax.iota(jnp.int32, 16)

    def group_body(g, carry):
        acc = jnp.zeros((16,), jnp.float32)
        for r in range(16):
            row = g * 16 + r
            p = jnp.zeros((16,), jnp.float32)
            for c in range(_EMBED // 16):
                u = urows[row, pl.ds(c * 16, 16)]
                v = irows[row, pl.ds(c * 16, 16)]
                p = p + u * v
            acc = jnp.where(lane == r, jnp.sum(p), acc)
        out_v[pl.ds(g * 16, 16)] = acc
        return carry

    lax.fori_loop(0, _BPW // 16, group_body, 0)

    pltpu.sync_copy(out_v, out_hbm.at[pl.ds(base, _BPW)])


def kernel(user_ids, item_ids, user_table, item_table):
    return _sc_kernel(user_ids, item_ids, user_table, item_table)
